# baseline (speedup 1.0000x reference)
import functools

import jax
import jax.numpy as jnp
from jax import lax
from jax.experimental import pallas as pl
from jax.experimental.pallas import tpu as pltpu
from jax.experimental.pallas import tpu_sc as plsc

N = 1048576
SUBJ = 1000000
D = 8
NC = 2
NS = 16
L = 16
NW = NC * NS
PER_W = N // NW
C = 2048
NCHUNK = PER_W // C

_mesh = plsc.VectorSubcoreMesh(core_axis_name="c", subcore_axis_name="s")


@functools.partial(
    pl.kernel,
    mesh=_mesh,
    compiler_params=pltpu.CompilerParams(use_tc_tiling_on_sc=False),
    out_type=jax.ShapeDtypeStruct((NW, NCHUNK, C, D), jnp.float32),
    scratch_types=[
        pltpu.VMEM((C,), jnp.int32),
        pltpu.VMEM((C, D), jnp.float32),
        pltpu.SemaphoreType.DMA,
    ],
)
def _gather_rows(sub_hbm, tab_hbm, g_hbm, idx_v, p_v, sem):
    wid = lax.axis_index("s") * NC + lax.axis_index("c")

    def chunk_body(c, carry):
        pltpu.sync_copy(sub_hbm.at[wid, c], idx_v)
        pltpu.async_copy(tab_hbm.at[idx_v], p_v, sem).wait()
        pltpu.sync_copy(p_v, g_hbm.at[wid, c])
        return carry

    lax.fori_loop(0, NCHUNK, chunk_body, 0)


@functools.partial(
    pl.kernel,
    mesh=_mesh,
    compiler_params=pltpu.CompilerParams(
        use_tc_tiling_on_sc=False, needs_layout_passes=False),
    out_type=jax.ShapeDtypeStruct((NW, L), jnp.float32),
    scratch_types=[
        pltpu.VMEM((C * D,), jnp.float32),
        pltpu.VMEM((C,), jnp.float32),
        pltpu.VMEM((C,), jnp.float32),
        pltpu.VMEM((C,), jnp.float32),
        pltpu.VMEM((C,), jnp.float32),
        pltpu.VMEM((L,), jnp.float32),
        pltpu.SemaphoreType.DMA,
    ],
)
def _rmse_partials(gf_hbm, y_hbm, j_hbm, k1_hbm, k2_hbm,
                   out_hbm,
                   pf_v, y_v, j_v, k1_v, k2_v,
                   acc_v, sem):
    wid = lax.axis_index("s") * NC + lax.axis_index("c")
    lane = lax.iota(jnp.int32, L)

    def chunk_body(c, acc):
        pltpu.sync_copy(gf_hbm.at[wid, c], pf_v)
        pltpu.sync_copy(y_hbm.at[wid, c], y_v)
        pltpu.sync_copy(j_hbm.at[wid, c], j_v)
        pltpu.sync_copy(k1_hbm.at[wid, c], k1_v)
        pltpu.sync_copy(k2_hbm.at[wid, c], k2_v)

        def vec_body(v, acc_v16):
            s = pl.ds(v * L, L)
            base = (v * L + lane) * D
            a = plsc.load_gather(pf_v, [base])
            u = plsc.load_gather(pf_v, [base + 1])
            lm = plsc.load_gather(pf_v, [base + 2])
            g1 = plsc.load_gather(pf_v, [base + 3])
            g2 = plsc.load_gather(pf_v, [base + 4])
            a = jnp.maximum(a, 0.0)
            u = jnp.maximum(u, 0.0)
            lm = 0.2 / (1.0 + jnp.exp(-lm))
            g1 = 1.0 / (1.0 + jnp.exp(-g1))
            g2 = 1.0 / (1.0 + jnp.exp(-g2))
            t = j_v[s] + g1 * k1_v[s] + g2 * k2_v[s]
            mu = a - u * jnp.exp(-lm * t)
            resid = y_v[s] - mu
            return acc_v16 + resid * resid

        return lax.fori_loop(0, C // L, vec_body, acc)

    acc = lax.fori_loop(0, NCHUNK, chunk_body, jnp.zeros((L,), jnp.float32))
    acc_v[...] = acc
    pltpu.sync_copy(acc_v, out_hbm.at[wid])


def kernel(y, j, k1, k2, sub, A, U, Lambda, Gamma1, Gamma2):
    z = jnp.zeros_like(A)
    packed = jnp.stack([A, U, Lambda, Gamma1, Gamma2, z, z, z], axis=1)
    shp = (NW, NCHUNK, C)
    g = _gather_rows(sub.astype(jnp.int32).reshape(shp), packed)
    partials = _rmse_partials(
        g.reshape(NW, NCHUNK, C * D),
        y.reshape(shp), j.reshape(shp), k1.reshape(shp), k2.reshape(shp),
    )
    return jnp.sqrt(jnp.sum(partials) / N)


# speedup vs baseline: 5.4404x; 5.4404x over previous
import functools

import jax
import jax.numpy as jnp
from jax import lax
from jax.experimental import pallas as pl
from jax.experimental.pallas import tpu as pltpu
from jax.experimental.pallas import tpu_sc as plsc

N = 1048576
NC = 2
NS = 16
L = 16
NW = NC * NS
PER_W = N // NW
C = 2048
NCHUNK = PER_W // C
NBUF = 2

_mesh = plsc.VectorSubcoreMesh(core_axis_name="c", subcore_axis_name="s")


def _buf_set():
    return [
        pltpu.VMEM((C,), jnp.int32),
        pltpu.VMEM((C,), jnp.float32),
        pltpu.VMEM((C,), jnp.float32),
        pltpu.VMEM((C,), jnp.float32),
        pltpu.VMEM((C,), jnp.float32),
        pltpu.VMEM((C,), jnp.float32),
        pltpu.VMEM((C,), jnp.float32),
        pltpu.VMEM((C,), jnp.float32),
        pltpu.VMEM((C,), jnp.float32),
        pltpu.VMEM((C,), jnp.float32),
        pltpu.SemaphoreType.DMA,
    ]


@functools.partial(
    pl.kernel,
    mesh=_mesh,
    out_type=jax.ShapeDtypeStruct((NW, L), jnp.float32),
    scratch_types=[
        *_buf_set(),
        *_buf_set(),
        pltpu.VMEM((L,), jnp.float32),
    ],
)
def _rmse_partials(y_hbm, j_hbm, k1_hbm, k2_hbm, sub_hbm,
                   a_hbm, u_hbm, lam_hbm, g1_hbm, g2_hbm,
                   out_hbm,
                   *scratch):
    bufs = [scratch[:11], scratch[11:22]]
    acc_v = scratch[22]
    wid = lax.axis_index("s") * NC + lax.axis_index("c")
    base = wid * PER_W
    tabs = (a_hbm, u_hbm, lam_hbm, g1_hbm, g2_hbm)

    def fire(c, buf):
        idx_v, y_v, j_v, k1_v, k2_v = buf[:5]
        sem = buf[10]
        off = base + c * C
        pltpu.sync_copy(sub_hbm.at[pl.ds(off, C)], idx_v)
        pltpu.sync_copy(y_hbm.at[pl.ds(off, C)], y_v)
        pltpu.sync_copy(j_hbm.at[pl.ds(off, C)], j_v)
        pltpu.sync_copy(k1_hbm.at[pl.ds(off, C)], k1_v)
        pltpu.sync_copy(k2_hbm.at[pl.ds(off, C)], k2_v)
        for t in range(5):
            pltpu.async_copy(tabs[t].at[idx_v], buf[5 + t], sem)

    def drain(buf):
        idx_v = buf[0]
        sem = buf[10]
        for t in range(5):
            pltpu.make_async_copy(tabs[t].at[idx_v], buf[5 + t], sem).wait()

    def compute(buf, acc):
        _, y_v, j_v, k1_v, k2_v, a_v, u_v, lam_v, g1_v, g2_v, _ = buf

        def vec_body(v, acc_v16):
            s = pl.ds(v * L, L)
            a = jnp.maximum(a_v[s], 0.0)
            u = jnp.maximum(u_v[s], 0.0)
            lm = 0.2 / (1.0 + jnp.exp(-lam_v[s]))
            g1 = 1.0 / (1.0 + jnp.exp(-g1_v[s]))
            g2 = 1.0 / (1.0 + jnp.exp(-g2_v[s]))
            t = j_v[s] + g1 * k1_v[s] + g2 * k2_v[s]
            mu = a - u * jnp.exp(-lm * t)
            resid = y_v[s] - mu
            return acc_v16 + resid * resid

        return lax.fori_loop(0, C // L, vec_body, acc)

    acc = jnp.zeros((L,), jnp.float32)
    fire(0, bufs[0])
    for c in range(NCHUNK):
        cur = bufs[c % NBUF]
        if c + 1 < NCHUNK:
            fire(c + 1, bufs[(c + 1) % NBUF])
        drain(cur)
        acc = compute(cur, acc)
    acc_v[...] = acc
    pltpu.sync_copy(acc_v, out_hbm.at[wid])


def kernel(y, j, k1, k2, sub, A, U, Lambda, Gamma1, Gamma2):
    partials = _rmse_partials(y, j, k1, k2, sub.astype(jnp.int32),
                              A, U, Lambda, Gamma1, Gamma2)
    return jnp.sqrt(jnp.sum(partials) / N)
